# Initial kernel scaffold; baseline (speedup 1.0000x reference)
import jax
import jax.numpy as jnp
from jax.experimental import pallas as pl


def kernel(x, edge_index, W_mu, b_mu, W_logstd, b_logstd):
    raise NotImplementedError("write your pallas kernel here")


# speedup vs baseline: 18.2670x; 18.2670x over previous
import functools

import jax
import jax.numpy as jnp
from jax import lax
from jax.experimental import pallas as pl
from jax.experimental.pallas import tpu as pltpu
from jax.experimental.pallas import tpu_sc as plsc

NC = 2
NS = 16
NW = NC * NS
CHUNK = 128


def _sc_mesh():
    return plsc.VectorSubcoreMesh(
        core_axis_name="c", subcore_axis_name="s", num_cores=NC,
        num_subcores=NS)


def _fill_f32(ref, n_rows, n_cols, value):
    def row(j, _):
        def col(k, _):
            ref[j, pl.ds(k * 16, 16)] = jnp.full((16,), value, jnp.float32)
            return _
        return lax.fori_loop(0, n_cols // 16, col, _)
    lax.fori_loop(0, n_rows, row, None)


def _deg_body(ec, n_pad, dst_hbm, deg_hbm, dst_v, ones_v, zrow_v, deg_sh):
    c = lax.axis_index("c")
    s = lax.axis_index("s")
    wid = s * NC + c
    rpt = n_pad // NS

    _fill_f32(ones_v, 1, CHUNK, 1.0)
    _fill_f32(zrow_v, 1, rpt, 0.0)
    pltpu.sync_copy(zrow_v.at[0], deg_sh.at[pl.ds(s * rpt, rpt)])
    plsc.subcore_barrier()

    pltpu.sync_copy(dst_hbm.at[pl.ds(wid * ec, ec)], dst_v)

    def edge(j, _):
        pltpu.sync_copy(ones_v.at[0], deg_sh.at[dst_v.at[j]], add=True)
        return _
    lax.fori_loop(0, ec, edge, None)
    plsc.subcore_barrier()

    pltpu.sync_copy(deg_sh.at[pl.ds(s * rpt, rpt)],
                    deg_hbm.at[pl.ds(c * n_pad + s * rpt, rpt)])


def _sc_degree(dst_p, n_pad):
    ec = dst_p.shape[0] // NW
    kern = pl.kernel(
        functools.partial(_deg_body, ec, n_pad),
        out_type=jax.ShapeDtypeStruct((NC * n_pad,), jnp.float32),
        mesh=_sc_mesh(),
        scratch_types=[
            pltpu.VMEM((ec, CHUNK), jnp.int32),
            pltpu.VMEM((1, CHUNK), jnp.float32),
            pltpu.VMEM((1, n_pad // NS), jnp.float32),
            pltpu.VMEM_SHARED((n_pad,), jnp.float32),
        ],
    )
    return kern(dst_p)


def _agg_body(ec, n_pad, z_hbm, src_hbm, dst_hbm, agg_hbm,
              src_v, dst_v, rows_v, zrow_v, agg_sh, sem):
    c = lax.axis_index("c")
    s = lax.axis_index("s")
    wid = s * NC + c
    rpt = n_pad // NS
    q = rpt // 8

    _fill_f32(zrow_v, q, 128, 0.0)

    def zero(i, _):
        pltpu.sync_copy(zrow_v, agg_sh.at[pl.ds(s * rpt + i * q, q)])
        return _
    lax.fori_loop(0, 8, zero, None)

    pltpu.sync_copy(src_hbm.at[pl.ds(wid * ec, ec)], src_v)
    pltpu.sync_copy(dst_hbm.at[pl.ds(wid * ec, ec)], dst_v)
    plsc.subcore_barrier()

    def edge(j, _):
        pltpu.async_copy(z_hbm.at[src_v.at[j]], rows_v, sem).wait()
        pltpu.sync_copy(rows_v, agg_sh.at[dst_v.at[j]], add=True)
        return _
    lax.fori_loop(0, ec, edge, None)
    plsc.subcore_barrier()

    def out(i, _):
        sl = pl.ds(s * rpt + i * q, q)
        pltpu.sync_copy(agg_sh.at[sl], zrow_v)
        pltpu.sync_copy(zrow_v, agg_hbm.at[c, sl])
        return _
    lax.fori_loop(0, 8, out, None)


def _sc_aggregate(z, src_p, dst_p, n_pad):
    ec = src_p.shape[0] // NW
    kern = pl.kernel(
        functools.partial(_agg_body, ec, n_pad),
        out_type=jax.ShapeDtypeStruct((NC, n_pad, 128), jnp.float32),
        mesh=_sc_mesh(),
        scratch_types=[
            pltpu.VMEM((ec, CHUNK), jnp.int32),
            pltpu.VMEM((ec, CHUNK), jnp.int32),
            pltpu.VMEM((CHUNK, 128), jnp.float32),
            pltpu.VMEM((n_pad // NS // 8, 128), jnp.float32),
            pltpu.VMEM_SHARED((n_pad, 128), jnp.float32),
            pltpu.SemaphoreType.DMA,
        ],
    )
    return kern(z, src_p, dst_p)


def _z_kernel(x_ref, w_ref, deg_ref, z_ref):
    i = pl.program_id(0)
    rb = x_ref.shape[0]
    dv = deg_ref[0, pl.ds(i * rb, rb)] + deg_ref[1, pl.ds(i * rb, rb)] + 1.0
    dinv = lax.rsqrt(dv)
    xw = jnp.dot(x_ref[...], w_ref[...], preferred_element_type=jnp.float32)
    z_ref[...] = xw * dinv[:, None]


def _tc_z(x_pad, w_cat, deg2, rb):
    n_pad, d_in = x_pad.shape
    grid = n_pad // rb
    return pl.pallas_call(
        _z_kernel,
        grid=(grid,),
        in_specs=[
            pl.BlockSpec((rb, d_in), lambda i: (i, 0)),
            pl.BlockSpec((d_in, 128), lambda i: (0, 0)),
            pl.BlockSpec((NC, n_pad), lambda i: (0, 0)),
        ],
        out_specs=pl.BlockSpec((rb, 128), lambda i: (i, 0)),
        out_shape=jax.ShapeDtypeStruct((n_pad, 128), jnp.float32),
    )(x_pad, w_cat, deg2)


def _out_kernel(agg_ref, z_ref, deg_ref, b_ref, out_ref):
    i = pl.program_id(0)
    rb = z_ref.shape[0]
    dv = deg_ref[0, pl.ds(i * rb, rb)] + deg_ref[1, pl.ds(i * rb, rb)] + 1.0
    dinv = lax.rsqrt(dv)
    ssum = agg_ref[0] + agg_ref[1] + z_ref[...]
    out_ref[...] = ssum * dinv[:, None] + b_ref[...]


def _tc_out(agg2, z, deg2, b_cat, rb):
    n_pad = z.shape[0]
    grid = n_pad // rb
    return pl.pallas_call(
        _out_kernel,
        grid=(grid,),
        in_specs=[
            pl.BlockSpec((NC, rb, 128), lambda i: (0, i, 0)),
            pl.BlockSpec((rb, 128), lambda i: (i, 0)),
            pl.BlockSpec((NC, n_pad), lambda i: (0, 0)),
            pl.BlockSpec((1, 128), lambda i: (0, 0)),
        ],
        out_specs=pl.BlockSpec((rb, 128), lambda i: (i, 0)),
        out_shape=jax.ShapeDtypeStruct((n_pad, 128), jnp.float32),
    )(agg2, z, deg2, b_cat)


def kernel(x, edge_index, W_mu, b_mu, W_logstd, b_logstd):
    n, d_in = x.shape
    d_out = W_mu.shape[1]
    e = edge_index.shape[1]

    n_pad = ((n + 1 + NS * 64 - 1) // (NS * 64)) * (NS * 64)
    ec = (e + NW * CHUNK - 1) // (NW * CHUNK)
    ec = ((ec + 7) // 8) * 8
    e_pad = ec * NW * CHUNK

    src = edge_index[0].astype(jnp.int32)
    dst = edge_index[1].astype(jnp.int32)
    pad = jnp.full((e_pad - e,), n, dtype=jnp.int32)
    src_p = jnp.concatenate([src, pad]).reshape(NW * ec, CHUNK)
    dst_p = jnp.concatenate([dst, pad]).reshape(NW * ec, CHUNK)

    x_pad = jnp.concatenate(
        [x, jnp.zeros((n_pad - n, d_in), dtype=x.dtype)], axis=0)
    w_cat = jnp.concatenate([W_mu, W_logstd], axis=1)
    b_cat = jnp.concatenate([b_mu, b_logstd]).reshape(1, 2 * d_out)

    rb = n_pad // 16

    deg2 = _sc_degree(dst_p, n_pad).reshape(NC, n_pad)
    z = _tc_z(x_pad, w_cat, deg2, rb)
    agg2 = _sc_aggregate(z, src_p, dst_p, n_pad)
    out_full = _tc_out(agg2, z, deg2, b_cat, rb)

    mu = out_full[:n, :d_out]
    logstd = out_full[:n, d_out:]
    return (mu, logstd)
